# Initial kernel scaffold; baseline (speedup 1.0000x reference)
import jax
import jax.numpy as jnp
from jax.experimental import pallas as pl


def kernel(x, edge_attr, edge_index, eW1, eb1, eW2, eb2, eW3, eb3, eg, ebt, nW1, nb1, nW2, nb2, nW3, nb3, ng, nbt):
    raise NotImplementedError("write your pallas kernel here")


# speedup vs baseline: 2.2911x; 2.2911x over previous
import functools

import jax
import jax.numpy as jnp
from jax import lax
from jax.experimental import pallas as pl
from jax.experimental.pallas import tpu as pltpu
from jax.experimental.pallas import tpu_sc as plsc

_N = 10000
_E = 320000
_D = 128

_NC = 2
_NS = 16
_NW = _NC * _NS
_C = 80

_EPW = _E // _NW
_KPW = _EPW // _C
_EPC = _E // _NC
_EPT = _EPC // _NS
_KPT = _EPT // _C
_NP = 10240
_RPT = _NP // _NS


def _sc_mesh():
    return plsc.VectorSubcoreMesh(core_axis_name="c", subcore_axis_name="s")


def _gather_pair(xs1, xs2, src_r, dst_r):
    @functools.partial(
        pl.kernel,
        mesh=_sc_mesh(),
        out_type=(
            jax.ShapeDtypeStruct((_E, _D), jnp.float32),
            jax.ShapeDtypeStruct((_E, _D), jnp.float32),
        ),
        scratch_types=[
            pltpu.VMEM((_KPW, _C), jnp.int32),
            pltpu.VMEM((_KPW, _C), jnp.int32),
            pltpu.VMEM((_C, _D), jnp.float32),
            pltpu.VMEM((_C, _D), jnp.float32),
            pltpu.SemaphoreType.DMA,
        ],
    )
    def k(xs1_h, xs2_h, src_h, dst_h, oa_h, ob_h, sidx, didx, bufa, bufb, sem):
        wid = lax.axis_index("s") * _NC + lax.axis_index("c")
        kbase = wid * _KPW
        pltpu.sync_copy(src_h.at[wid], sidx)
        pltpu.sync_copy(dst_h.at[wid], didx)

        def body(i, _):
            ca = pltpu.async_copy(xs1_h.at[sidx.at[i]], bufa, sem)
            cb = pltpu.async_copy(xs2_h.at[didx.at[i]], bufb, sem)
            ca.wait()
            cb.wait()
            ebase = (kbase + i) * _C
            pltpu.sync_copy(bufa, oa_h.at[pl.ds(ebase, _C)])
            pltpu.sync_copy(bufb, ob_h.at[pl.ds(ebase, _C)])
            return 0

        lax.fori_loop(0, _KPW, body, 0)

    return k(xs1, xs2, src_r, dst_r)


def _segment_sum(new_edge, dst_r, zeros_nd):
    @functools.partial(
        pl.kernel,
        mesh=_sc_mesh(),
        out_type=jax.ShapeDtypeStruct((_NC * _NP, _D), jnp.float32),
        scratch_types=[
            pltpu.VMEM((_KPT, _C), jnp.int32),
            pltpu.VMEM((_C, _D), jnp.float32),
            pltpu.VMEM_SHARED((_NP, _D), jnp.float32),
            pltpu.SemaphoreType.DMA,
        ],
    )
    def k(edge_h, dst_h, zer_h, out_h, didx, rows, agg_sh, sem):
        c = lax.axis_index("c")
        s = lax.axis_index("s")
        kbase = c * _EPC // _C + s * _KPT
        rbase = s * _RPT
        pltpu.sync_copy(zer_h.at[pl.ds(rbase, _RPT)], agg_sh.at[pl.ds(rbase, _RPT)])
        pltpu.sync_copy(dst_h.at[c, s], didx)
        plsc.subcore_barrier()

        def body(i, _):
            pltpu.sync_copy(edge_h.at[pl.ds((kbase + i) * _C, _C)], rows)
            pltpu.sync_copy(rows, agg_sh.at[didx.at[i]], add=True)
            return 0

        lax.fori_loop(0, _KPT, body, 0)
        plsc.subcore_barrier()
        pltpu.sync_copy(
            agg_sh.at[pl.ds(rbase, _RPT)], out_h.at[pl.ds(c * _NP + rbase, _RPT)]
        )

    return k(new_edge, dst_r, zeros_nd)


def _precompute(x, wcat):
    def body(x_ref, w_ref, o_ref):
        o_ref[...] = jnp.dot(
            x_ref[...], w_ref[...], preferred_element_type=jnp.float32
        )

    return pl.pallas_call(
        body,
        out_shape=jax.ShapeDtypeStruct((_N, 3 * _D), jnp.float32),
    )(x, wcat)


def _mlp_ln(h, w2_ref, w3_ref, b2_ref, b3_ref, g_ref, bt_ref):
    h = jnp.maximum(
        jnp.dot(h, w2_ref[...], preferred_element_type=jnp.float32) + b2_ref[...],
        0.0,
    )
    h = jnp.dot(h, w3_ref[...], preferred_element_type=jnp.float32) + b3_ref[...]
    m = jnp.mean(h, axis=1, keepdims=True)
    d = h - m
    v = jnp.mean(d * d, axis=1, keepdims=True)
    return d * lax.rsqrt(v + 1e-5) * g_ref[...] + bt_ref[...]


def _edge_mlp(ea, pa, pb, w1a, w2, w3, b1, b2, b3, g, bt):
    be = 512

    def body(ea_ref, pa_ref, pb_ref, w1_ref, w2_ref, w3_ref, b1_ref, b2_ref,
             b3_ref, g_ref, bt_ref, o_ref):
        h = (
            jnp.dot(ea_ref[...], w1_ref[...], preferred_element_type=jnp.float32)
            + pa_ref[...]
            + pb_ref[...]
            + b1_ref[...]
        )
        h = jnp.maximum(h, 0.0)
        o_ref[...] = _mlp_ln(h, w2_ref, w3_ref, b2_ref, b3_ref, g_ref, bt_ref)

    blk = lambda r: pl.BlockSpec((r, _D), lambda i: (i, 0))
    full = pl.BlockSpec((_D, _D), lambda i: (0, 0))
    vec = pl.BlockSpec((1, _D), lambda i: (0, 0))
    return pl.pallas_call(
        body,
        grid=(_E // be,),
        in_specs=[blk(be), blk(be), blk(be), full, full, full, vec, vec, vec,
                  vec, vec],
        out_specs=blk(be),
        out_shape=jax.ShapeDtypeStruct((_E, _D), jnp.float32),
    )(ea, pa, pb, w1a, w2, w3, b1, b2, b3, g, bt)


def _node_mlp(x, xn, a0, a1, w1b, w2, w3, b1, b2, b3, g, bt):
    bn = 1000

    def body(x_ref, xn_ref, a0_ref, a1_ref, w1_ref, w2_ref, w3_ref, b1_ref,
             b2_ref, b3_ref, g_ref, bt_ref, o_ref):
        agg = a0_ref[...] + a1_ref[...]
        h = (
            jnp.dot(agg, w1_ref[...], preferred_element_type=jnp.float32)
            + xn_ref[...]
            + b1_ref[...]
        )
        h = jnp.maximum(h, 0.0)
        ln = _mlp_ln(h, w2_ref, w3_ref, b2_ref, b3_ref, g_ref, bt_ref)
        o_ref[...] = x_ref[...] + ln

    blk = lambda r: pl.BlockSpec((r, _D), lambda i: (i, 0))
    full = pl.BlockSpec((_D, _D), lambda i: (0, 0))
    vec = pl.BlockSpec((1, _D), lambda i: (0, 0))
    return pl.pallas_call(
        body,
        grid=(_N // bn,),
        in_specs=[blk(bn), blk(bn), blk(bn), blk(bn), full, full, full, vec,
                  vec, vec, vec, vec],
        out_specs=blk(bn),
        out_shape=jax.ShapeDtypeStruct((_N, _D), jnp.float32),
    )(x, xn, a0, a1, w1b, w2, w3, b1, b2, b3, g, bt)


def kernel(x, edge_attr, edge_index, eW1, eb1, eW2, eb2, eW3, eb3, eg, ebt,
           nW1, nb1, nW2, nb2, nW3, nb3, ng, nbt):
    src_r = edge_index[0].reshape(_NW, _KPW, _C)
    dst_r = edge_index[1].reshape(_NW, _KPW, _C)
    dst_r4 = edge_index[1].reshape(_NC, _NS, _KPT, _C)

    row = lambda v: v.reshape(1, _D)

    wcat = jnp.concatenate(
        [eW1[_D : 2 * _D], eW1[2 * _D :], nW1[:_D]], axis=1
    )
    pre = _precompute(x, wcat)
    xs1 = pre[:, :_D]
    xs2 = pre[:, _D : 2 * _D]
    xn = pre[:, 2 * _D :]

    pa, pb = _gather_pair(xs1, xs2, src_r, dst_r)

    new_edge = _edge_mlp(
        edge_attr, pa, pb, eW1[:_D], eW2, eW3, row(eb1), row(eb2), row(eb3),
        row(eg), row(ebt)
    )

    zeros_nd = jnp.zeros((_NP, _D), jnp.float32)
    agg2 = _segment_sum(new_edge, dst_r4, zeros_nd)

    x_out = _node_mlp(
        x, xn, agg2[:_N], agg2[_NP : _NP + _N], nW1[_D:], nW2, nW3, row(nb1),
        row(nb2), row(nb3), row(ng), row(nbt)
    )
    return (x_out, new_edge)


# speedup vs baseline: 3.1508x; 1.3752x over previous
import functools

import jax
import jax.numpy as jnp
from jax import lax
from jax.experimental import pallas as pl
from jax.experimental.pallas import tpu as pltpu
from jax.experimental.pallas import tpu_sc as plsc

_N = 10000
_E = 320000
_D = 128

_NC = 2
_NS = 16
_NW = _NC * _NS
_C = 80

_EPW = _E // _NW
_KPW = _EPW // _C
_EPC = _E // _NC
_EPT = _EPC // _NS
_KPT = _EPT // _C
_NP = 10240
_RPT = _NP // _NS


def _sc_mesh():
    return plsc.VectorSubcoreMesh(core_axis_name="c", subcore_axis_name="s")


def _gather_pair(xs1, xs2, src_r, dst_r):
    @functools.partial(
        pl.kernel,
        mesh=_sc_mesh(),
        out_type=(
            jax.ShapeDtypeStruct((_E, _D), jnp.float32),
            jax.ShapeDtypeStruct((_E, _D), jnp.float32),
        ),
        scratch_types=[
            pltpu.VMEM((_KPW, _C), jnp.int32),
            pltpu.VMEM((_KPW, _C), jnp.int32),
            pltpu.VMEM((_C, _D), jnp.float32),
            pltpu.VMEM((_C, _D), jnp.float32),
            pltpu.SemaphoreType.DMA,
        ],
    )
    def k(xs1_h, xs2_h, src_h, dst_h, oa_h, ob_h, sidx, didx, bufa, bufb, sem):
        wid = lax.axis_index("s") * _NC + lax.axis_index("c")
        kbase = wid * _KPW
        pltpu.sync_copy(src_h.at[wid], sidx)
        pltpu.sync_copy(dst_h.at[wid], didx)

        def body(i, _):
            ca = pltpu.async_copy(xs1_h.at[sidx.at[i]], bufa, sem)
            cb = pltpu.async_copy(xs2_h.at[didx.at[i]], bufb, sem)
            ca.wait()
            cb.wait()
            ebase = (kbase + i) * _C
            pltpu.sync_copy(bufa, oa_h.at[pl.ds(ebase, _C)])
            pltpu.sync_copy(bufb, ob_h.at[pl.ds(ebase, _C)])
            return 0

        lax.fori_loop(0, _KPW, body, 0)

    return k(xs1, xs2, src_r, dst_r)


def _segment_sum(new_edge, dst_r, zeros_nd):
    @functools.partial(
        pl.kernel,
        mesh=_sc_mesh(),
        out_type=jax.ShapeDtypeStruct((_NC * _NP, _D), jnp.float32),
        scratch_types=[
            pltpu.VMEM((_KPT, _C), jnp.int32),
            pltpu.VMEM((_C, _D), jnp.float32),
            pltpu.VMEM_SHARED((_NP, _D), jnp.float32),
            pltpu.SemaphoreType.DMA,
        ],
    )
    def k(edge_h, dst_h, zer_h, out_h, didx, rows, agg_sh, sem):
        c = lax.axis_index("c")
        s = lax.axis_index("s")
        kbase = c * _EPC // _C + s * _KPT
        rbase = s * _RPT
        pltpu.sync_copy(zer_h.at[pl.ds(rbase, _RPT)], agg_sh.at[pl.ds(rbase, _RPT)])
        pltpu.sync_copy(dst_h.at[c, s], didx)
        plsc.subcore_barrier()

        def body(i, _):
            pltpu.sync_copy(edge_h.at[pl.ds((kbase + i) * _C, _C)], rows)
            pltpu.sync_copy(rows, agg_sh.at[didx.at[i]], add=True)
            return 0

        lax.fori_loop(0, _KPT, body, 0)
        plsc.subcore_barrier()
        pltpu.sync_copy(
            agg_sh.at[pl.ds(rbase, _RPT)], out_h.at[pl.ds(c * _NP + rbase, _RPT)]
        )

    return k(new_edge, dst_r, zeros_nd)


def _precompute(x, wcat):
    def body(x_ref, w_ref, o_ref):
        o_ref[...] = jnp.dot(
            x_ref[...], w_ref[...], preferred_element_type=jnp.float32
        )

    return pl.pallas_call(
        body,
        out_shape=jax.ShapeDtypeStruct((_N, 3 * _D), jnp.float32),
    )(x, wcat)


def _mlp_ln(h, w2_ref, w3_ref, b2_ref, b3_ref, g_ref, bt_ref):
    h = jnp.maximum(
        jnp.dot(h, w2_ref[...], preferred_element_type=jnp.float32) + b2_ref[...],
        0.0,
    )
    h = jnp.dot(h, w3_ref[...], preferred_element_type=jnp.float32) + b3_ref[...]
    m = jnp.mean(h, axis=1, keepdims=True)
    d = h - m
    v = jnp.mean(d * d, axis=1, keepdims=True)
    return d * lax.rsqrt(v + 1e-5) * g_ref[...] + bt_ref[...]


def _edge_mlp(ea, pa, pb, w1a, w2, w3, b1, b2, b3, g, bt):
    be = 2000

    def body(ea_ref, pa_ref, pb_ref, w1_ref, w2_ref, w3_ref, b1_ref, b2_ref,
             b3_ref, g_ref, bt_ref, o_ref):
        h = (
            jnp.dot(ea_ref[...], w1_ref[...], preferred_element_type=jnp.float32)
            + pa_ref[...]
            + pb_ref[...]
            + b1_ref[...]
        )
        h = jnp.maximum(h, 0.0)
        o_ref[...] = _mlp_ln(h, w2_ref, w3_ref, b2_ref, b3_ref, g_ref, bt_ref)

    blk = lambda r: pl.BlockSpec((r, _D), lambda i: (i, 0))
    full = pl.BlockSpec((_D, _D), lambda i: (0, 0))
    vec = pl.BlockSpec((1, _D), lambda i: (0, 0))
    return pl.pallas_call(
        body,
        grid=(_E // be,),
        in_specs=[blk(be), blk(be), blk(be), full, full, full, vec, vec, vec,
                  vec, vec],
        out_specs=blk(be),
        out_shape=jax.ShapeDtypeStruct((_E, _D), jnp.float32),
    )(ea, pa, pb, w1a, w2, w3, b1, b2, b3, g, bt)


def _node_mlp(x, xn, a0, a1, w1b, w2, w3, b1, b2, b3, g, bt):
    bn = 1000

    def body(x_ref, xn_ref, a0_ref, a1_ref, w1_ref, w2_ref, w3_ref, b1_ref,
             b2_ref, b3_ref, g_ref, bt_ref, o_ref):
        agg = a0_ref[...] + a1_ref[...]
        h = (
            jnp.dot(agg, w1_ref[...], preferred_element_type=jnp.float32)
            + xn_ref[...]
            + b1_ref[...]
        )
        h = jnp.maximum(h, 0.0)
        ln = _mlp_ln(h, w2_ref, w3_ref, b2_ref, b3_ref, g_ref, bt_ref)
        o_ref[...] = x_ref[...] + ln

    blk = lambda r: pl.BlockSpec((r, _D), lambda i: (i, 0))
    full = pl.BlockSpec((_D, _D), lambda i: (0, 0))
    vec = pl.BlockSpec((1, _D), lambda i: (0, 0))
    return pl.pallas_call(
        body,
        grid=(_N // bn,),
        in_specs=[blk(bn), blk(bn), blk(bn), blk(bn), full, full, full, vec,
                  vec, vec, vec, vec],
        out_specs=blk(bn),
        out_shape=jax.ShapeDtypeStruct((_N, _D), jnp.float32),
    )(x, xn, a0, a1, w1b, w2, w3, b1, b2, b3, g, bt)


def kernel(x, edge_attr, edge_index, eW1, eb1, eW2, eb2, eW3, eb3, eg, ebt,
           nW1, nb1, nW2, nb2, nW3, nb3, ng, nbt):
    src_r = edge_index[0].reshape(_NW, _KPW, _C)
    dst_r = edge_index[1].reshape(_NW, _KPW, _C)
    dst_r4 = edge_index[1].reshape(_NC, _NS, _KPT, _C)

    row = lambda v: v.reshape(1, _D)

    wcat = jnp.concatenate(
        [eW1[_D : 2 * _D], eW1[2 * _D :], nW1[:_D]], axis=1
    )
    pre = _precompute(x, wcat)
    xs1 = pre[:, :_D]
    xs2 = pre[:, _D : 2 * _D]
    xn = pre[:, 2 * _D :]

    pa, pb = _gather_pair(xs1, xs2, src_r, dst_r)

    new_edge = _edge_mlp(
        edge_attr, pa, pb, eW1[:_D], eW2, eW3, row(eb1), row(eb2), row(eb3),
        row(eg), row(ebt)
    )

    zeros_nd = jnp.zeros((_NP, _D), jnp.float32)
    agg2 = _segment_sum(new_edge, dst_r4, zeros_nd)

    x_out = _node_mlp(
        x, xn, agg2[:_N], agg2[_NP : _NP + _N], nW1[_D:], nW2, nW3, row(nb1),
        row(nb2), row(nb3), row(ng), row(nbt)
    )
    return (x_out, new_edge)


# speedup vs baseline: 3.3774x; 1.0719x over previous
import functools

import jax
import jax.numpy as jnp
from jax import lax
from jax.experimental import pallas as pl
from jax.experimental.pallas import tpu as pltpu
from jax.experimental.pallas import tpu_sc as plsc

_N = 10000
_E = 320000
_D = 128

_NC = 2
_NS = 16
_NW = _NC * _NS
_C = 80

_EPW = _E // _NW
_KPW = _EPW // _C
_EPC = _E // _NC
_EPT = _EPC // _NS
_KPT = _EPT // _C
_NP = 10240
_RPT = _NP // _NS


def _sc_mesh():
    return plsc.VectorSubcoreMesh(core_axis_name="c", subcore_axis_name="s")


def _gather_pair(xs1, xs2, src_r, dst_r):
    @functools.partial(
        pl.kernel,
        mesh=_sc_mesh(),
        out_type=(
            jax.ShapeDtypeStruct((_E, _D), jnp.float32),
            jax.ShapeDtypeStruct((_E, _D), jnp.float32),
        ),
        scratch_types=[
            pltpu.VMEM((_KPW, _C), jnp.int32),
            pltpu.VMEM((_KPW, _C), jnp.int32),
            pltpu.VMEM((_C, _D), jnp.float32),
            pltpu.VMEM((_C, _D), jnp.float32),
            pltpu.SemaphoreType.DMA,
        ],
    )
    def k(xs1_h, xs2_h, src_h, dst_h, oa_h, ob_h, sidx, didx, bufa, bufb, sem):
        wid = lax.axis_index("s") * _NC + lax.axis_index("c")
        kbase = wid * _KPW
        pltpu.sync_copy(src_h.at[wid], sidx)
        pltpu.sync_copy(dst_h.at[wid], didx)

        def body(i, _):
            ca = pltpu.async_copy(xs1_h.at[sidx.at[i]], bufa, sem)
            cb = pltpu.async_copy(xs2_h.at[didx.at[i]], bufb, sem)
            ca.wait()
            cb.wait()
            ebase = (kbase + i) * _C
            pltpu.sync_copy(bufa, oa_h.at[pl.ds(ebase, _C)])
            pltpu.sync_copy(bufb, ob_h.at[pl.ds(ebase, _C)])
            return 0

        lax.fori_loop(0, _KPW, body, 0)

    return k(xs1, xs2, src_r, dst_r)


def _segment_sum(new_edge, dst_r, zeros_nd):
    @functools.partial(
        pl.kernel,
        mesh=_sc_mesh(),
        out_type=jax.ShapeDtypeStruct((_NC * _NP, _D), jnp.float32),
        scratch_types=[
            pltpu.VMEM((_KPT, _C), jnp.int32),
            pltpu.VMEM((_C, _D), jnp.float32),
            pltpu.VMEM_SHARED((_NP, _D), jnp.float32),
            pltpu.SemaphoreType.DMA,
        ],
    )
    def k(edge_h, dst_h, zer_h, out_h, didx, rows, agg_sh, sem):
        c = lax.axis_index("c")
        s = lax.axis_index("s")
        kbase = c * _EPC // _C + s * _KPT
        rbase = s * _RPT
        pltpu.sync_copy(zer_h.at[pl.ds(rbase, _RPT)], agg_sh.at[pl.ds(rbase, _RPT)])
        pltpu.sync_copy(dst_h.at[c, s], didx)
        plsc.subcore_barrier()

        def body(i, _):
            pltpu.sync_copy(edge_h.at[pl.ds((kbase + i) * _C, _C)], rows)
            pltpu.sync_copy(rows, agg_sh.at[didx.at[i]], add=True)
            return 0

        lax.fori_loop(0, _KPT, body, 0)
        plsc.subcore_barrier()
        pltpu.sync_copy(
            agg_sh.at[pl.ds(rbase, _RPT)], out_h.at[pl.ds(c * _NP + rbase, _RPT)]
        )

    return k(new_edge, dst_r, zeros_nd)


def _precompute(x, wcat):
    def body(x_ref, w_ref, o_ref):
        o_ref[...] = jnp.dot(
            x_ref[...], w_ref[...], preferred_element_type=jnp.float32
        )

    return pl.pallas_call(
        body,
        out_shape=jax.ShapeDtypeStruct((_N, 3 * _D), jnp.float32),
    )(x, wcat)


def _mlp_ln(h, w2_ref, w3_ref, b2_ref, b3_ref, g_ref, bt_ref):
    h = jnp.maximum(
        jnp.dot(h, w2_ref[...], preferred_element_type=jnp.float32) + b2_ref[...],
        0.0,
    )
    h = jnp.dot(h, w3_ref[...], preferred_element_type=jnp.float32) + b3_ref[...]
    m = jnp.mean(h, axis=1, keepdims=True)
    d = h - m
    v = jnp.mean(d * d, axis=1, keepdims=True)
    return d * lax.rsqrt(v + 1e-5) * g_ref[...] + bt_ref[...]


def _edge_mlp(ea, pa, pb, w1a, w2, w3, b1, b2, b3, g, bt):
    be = 4000

    def body(ea_ref, pa_ref, pb_ref, w1_ref, w2_ref, w3_ref, b1_ref, b2_ref,
             b3_ref, g_ref, bt_ref, o_ref):
        h = (
            jnp.dot(ea_ref[...], w1_ref[...], preferred_element_type=jnp.float32)
            + pa_ref[...]
            + pb_ref[...]
            + b1_ref[...]
        )
        h = jnp.maximum(h, 0.0)
        o_ref[...] = _mlp_ln(h, w2_ref, w3_ref, b2_ref, b3_ref, g_ref, bt_ref)

    blk = lambda r: pl.BlockSpec((r, _D), lambda i: (i, 0))
    full = pl.BlockSpec((_D, _D), lambda i: (0, 0))
    vec = pl.BlockSpec((1, _D), lambda i: (0, 0))
    return pl.pallas_call(
        body,
        grid=(_E // be,),
        in_specs=[blk(be), blk(be), blk(be), full, full, full, vec, vec, vec,
                  vec, vec],
        out_specs=blk(be),
        out_shape=jax.ShapeDtypeStruct((_E, _D), jnp.float32),
    )(ea, pa, pb, w1a, w2, w3, b1, b2, b3, g, bt)


def _node_mlp(x, xn, a0, a1, w1b, w2, w3, b1, b2, b3, g, bt):
    bn = 1000

    def body(x_ref, xn_ref, a0_ref, a1_ref, w1_ref, w2_ref, w3_ref, b1_ref,
             b2_ref, b3_ref, g_ref, bt_ref, o_ref):
        agg = a0_ref[...] + a1_ref[...]
        h = (
            jnp.dot(agg, w1_ref[...], preferred_element_type=jnp.float32)
            + xn_ref[...]
            + b1_ref[...]
        )
        h = jnp.maximum(h, 0.0)
        ln = _mlp_ln(h, w2_ref, w3_ref, b2_ref, b3_ref, g_ref, bt_ref)
        o_ref[...] = x_ref[...] + ln

    blk = lambda r: pl.BlockSpec((r, _D), lambda i: (i, 0))
    full = pl.BlockSpec((_D, _D), lambda i: (0, 0))
    vec = pl.BlockSpec((1, _D), lambda i: (0, 0))
    return pl.pallas_call(
        body,
        grid=(_N // bn,),
        in_specs=[blk(bn), blk(bn), blk(bn), blk(bn), full, full, full, vec,
                  vec, vec, vec, vec],
        out_specs=blk(bn),
        out_shape=jax.ShapeDtypeStruct((_N, _D), jnp.float32),
    )(x, xn, a0, a1, w1b, w2, w3, b1, b2, b3, g, bt)


def kernel(x, edge_attr, edge_index, eW1, eb1, eW2, eb2, eW3, eb3, eg, ebt,
           nW1, nb1, nW2, nb2, nW3, nb3, ng, nbt):
    src_r = edge_index[0].reshape(_NW, _KPW, _C)
    dst_r = edge_index[1].reshape(_NW, _KPW, _C)
    dst_r4 = edge_index[1].reshape(_NC, _NS, _KPT, _C)

    row = lambda v: v.reshape(1, _D)

    wcat = jnp.concatenate(
        [eW1[_D : 2 * _D], eW1[2 * _D :], nW1[:_D]], axis=1
    )
    pre = _precompute(x, wcat)
    xs1 = pre[:, :_D]
    xs2 = pre[:, _D : 2 * _D]
    xn = pre[:, 2 * _D :]

    pa, pb = _gather_pair(xs1, xs2, src_r, dst_r)

    new_edge = _edge_mlp(
        edge_attr, pa, pb, eW1[:_D], eW2, eW3, row(eb1), row(eb2), row(eb3),
        row(eg), row(ebt)
    )

    zeros_nd = jnp.zeros((_NP, _D), jnp.float32)
    agg2 = _segment_sum(new_edge, dst_r4, zeros_nd)

    x_out = _node_mlp(
        x, xn, agg2[:_N], agg2[_NP : _NP + _N], nW1[_D:], nW2, nW3, row(nb1),
        row(nb2), row(nb3), row(ng), row(nbt)
    )
    return (x_out, new_edge)


# speedup vs baseline: 4.1366x; 1.2248x over previous
import functools

import jax
import jax.numpy as jnp
from jax import lax
from jax.experimental import pallas as pl
from jax.experimental.pallas import tpu as pltpu
from jax.experimental.pallas import tpu_sc as plsc

_N = 10000
_E = 320000
_D = 128

_NC = 2
_NS = 16
_NW = _NC * _NS
_C = 80

_EPW = _E // _NW
_KPW = _EPW // _C
_EPC = _E // _NC
_EPT = _EPC // _NS
_KPT = _EPT // _C
_NP = 10240
_RPT = _NP // _NS


def _sc_mesh():
    return plsc.VectorSubcoreMesh(core_axis_name="c", subcore_axis_name="s")


def _gather_sum(xs1, xs2, src_r, dst_r):
    @functools.partial(
        pl.kernel,
        mesh=_sc_mesh(),
        out_type=jax.ShapeDtypeStruct((_E, _D), jnp.float32),
        scratch_types=[
            pltpu.VMEM((_KPW, _C), jnp.int32),
            pltpu.VMEM((_KPW, _C), jnp.int32),
            pltpu.VMEM((_C, _D), jnp.float32),
            pltpu.VMEM((_C, _D), jnp.float32),
            pltpu.VMEM((_C, _D), jnp.float32),
            pltpu.VMEM((_C, _D), jnp.float32),
            pltpu.SemaphoreType.DMA,
            pltpu.SemaphoreType.DMA,
            pltpu.SemaphoreType.DMA,
            pltpu.SemaphoreType.DMA,
        ],
    )
    def k(xs1_h, xs2_h, src_h, dst_h, o_h, sidx, didx, a0, b0, a1, b1,
          g0, g1, o0, o1):
        wid = lax.axis_index("s") * _NC + lax.axis_index("c")
        kbase = wid * _KPW
        pltpu.sync_copy(src_h.at[wid], sidx)
        pltpu.sync_copy(dst_h.at[wid], didx)

        def start(i, a, b, g):
            pltpu.async_copy(xs1_h.at[sidx.at[i]], a, g)
            pltpu.async_copy(xs2_h.at[didx.at[i]], b, g)

        def fin(i, a, b, g, o):
            pltpu.make_async_copy(xs1_h.at[sidx.at[i]], a, g).wait()
            pltpu.make_async_copy(xs2_h.at[didx.at[i]], b, g).wait()

            def add_r(r, _):
                for j in range(8):
                    sl = pl.ds(j * 16, 16)
                    a[r, sl] = a[r, sl] + b[r, sl]
                return 0

            lax.fori_loop(0, _C, add_r, 0)
            pltpu.async_copy(a, o_h.at[pl.ds((kbase + i) * _C, _C)], o)

        def drain(i, a, o):
            pltpu.make_async_copy(a, o_h.at[pl.ds((kbase + i) * _C, _C)], o).wait()

        start(0, a0, b0, g0)

        def body(ii, _):
            i = 2 * ii

            @pl.when(i + 1 < _KPW)
            def _():
                @pl.when(i >= 1)
                def _():
                    drain(i - 1, a1, o1)

                start(i + 1, a1, b1, g1)

            fin(i, a0, b0, g0, o0)

            @pl.when(i + 2 < _KPW)
            def _():
                drain(i, a0, o0)
                start(i + 2, a0, b0, g0)

            @pl.when(i + 1 < _KPW)
            def _():
                fin(i + 1, a1, b1, g1, o1)

            return 0

        lax.fori_loop(0, (_KPW + 1) // 2, body, 0)
        drain(_KPW - 2, a1, o1)
        drain(_KPW - 1, a0, o0)

    return k(xs1, xs2, src_r, dst_r)


def _segment_sum(new_edge, dst_r, zeros_nd):
    @functools.partial(
        pl.kernel,
        mesh=_sc_mesh(),
        out_type=jax.ShapeDtypeStruct((_NC * _NP, _D), jnp.float32),
        scratch_types=[
            pltpu.VMEM((_KPT, _C), jnp.int32),
            pltpu.VMEM((_C, _D), jnp.float32),
            pltpu.VMEM_SHARED((_NP, _D), jnp.float32),
            pltpu.SemaphoreType.DMA,
        ],
    )
    def k(edge_h, dst_h, zer_h, out_h, didx, rows, agg_sh, sem):
        c = lax.axis_index("c")
        s = lax.axis_index("s")
        kbase = c * _EPC // _C + s * _KPT
        rbase = s * _RPT
        pltpu.sync_copy(zer_h.at[pl.ds(rbase, _RPT)], agg_sh.at[pl.ds(rbase, _RPT)])
        pltpu.sync_copy(dst_h.at[c, s], didx)
        plsc.subcore_barrier()

        def body(i, _):
            pltpu.sync_copy(edge_h.at[pl.ds((kbase + i) * _C, _C)], rows)
            pltpu.sync_copy(rows, agg_sh.at[didx.at[i]], add=True)
            return 0

        lax.fori_loop(0, _KPT, body, 0)
        plsc.subcore_barrier()
        pltpu.sync_copy(
            agg_sh.at[pl.ds(rbase, _RPT)], out_h.at[pl.ds(c * _NP + rbase, _RPT)]
        )

    return k(new_edge, dst_r, zeros_nd)


def _precompute(x, wcat):
    def body(x_ref, w_ref, o_ref):
        o_ref[...] = jnp.dot(
            x_ref[...], w_ref[...], preferred_element_type=jnp.float32
        )

    return pl.pallas_call(
        body,
        out_shape=jax.ShapeDtypeStruct((_N, 3 * _D), jnp.float32),
    )(x, wcat)


def _mlp_ln(h, w2_ref, w3_ref, b2_ref, b3_ref, g_ref, bt_ref):
    h = jnp.maximum(
        jnp.dot(h, w2_ref[...], preferred_element_type=jnp.float32) + b2_ref[...],
        0.0,
    )
    h = jnp.dot(h, w3_ref[...], preferred_element_type=jnp.float32) + b3_ref[...]
    m = jnp.mean(h, axis=1, keepdims=True)
    d = h - m
    v = jnp.mean(d * d, axis=1, keepdims=True)
    return d * lax.rsqrt(v + 1e-5) * g_ref[...] + bt_ref[...]


def _edge_mlp(ea, pa, w1a, w2, w3, b1, b2, b3, g, bt):
    be = 4000

    def body(ea_ref, pa_ref, w1_ref, w2_ref, w3_ref, b1_ref, b2_ref,
             b3_ref, g_ref, bt_ref, o_ref):
        h = (
            jnp.dot(ea_ref[...], w1_ref[...], preferred_element_type=jnp.float32)
            + pa_ref[...]
            + b1_ref[...]
        )
        h = jnp.maximum(h, 0.0)
        o_ref[...] = _mlp_ln(h, w2_ref, w3_ref, b2_ref, b3_ref, g_ref, bt_ref)

    blk = lambda r: pl.BlockSpec((r, _D), lambda i: (i, 0))
    full = pl.BlockSpec((_D, _D), lambda i: (0, 0))
    vec = pl.BlockSpec((1, _D), lambda i: (0, 0))
    return pl.pallas_call(
        body,
        grid=(_E // be,),
        in_specs=[blk(be), blk(be), full, full, full, vec, vec, vec,
                  vec, vec],
        out_specs=blk(be),
        out_shape=jax.ShapeDtypeStruct((_E, _D), jnp.float32),
    )(ea, pa, w1a, w2, w3, b1, b2, b3, g, bt)


def _node_mlp(x, xn, a0, a1, w1b, w2, w3, b1, b2, b3, g, bt):
    bn = 1000

    def body(x_ref, xn_ref, a0_ref, a1_ref, w1_ref, w2_ref, w3_ref, b1_ref,
             b2_ref, b3_ref, g_ref, bt_ref, o_ref):
        agg = a0_ref[...] + a1_ref[...]
        h = (
            jnp.dot(agg, w1_ref[...], preferred_element_type=jnp.float32)
            + xn_ref[...]
            + b1_ref[...]
        )
        h = jnp.maximum(h, 0.0)
        ln = _mlp_ln(h, w2_ref, w3_ref, b2_ref, b3_ref, g_ref, bt_ref)
        o_ref[...] = x_ref[...] + ln

    blk = lambda r: pl.BlockSpec((r, _D), lambda i: (i, 0))
    full = pl.BlockSpec((_D, _D), lambda i: (0, 0))
    vec = pl.BlockSpec((1, _D), lambda i: (0, 0))
    return pl.pallas_call(
        body,
        grid=(_N // bn,),
        in_specs=[blk(bn), blk(bn), blk(bn), blk(bn), full, full, full, vec,
                  vec, vec, vec, vec],
        out_specs=blk(bn),
        out_shape=jax.ShapeDtypeStruct((_N, _D), jnp.float32),
    )(x, xn, a0, a1, w1b, w2, w3, b1, b2, b3, g, bt)


def kernel(x, edge_attr, edge_index, eW1, eb1, eW2, eb2, eW3, eb3, eg, ebt,
           nW1, nb1, nW2, nb2, nW3, nb3, ng, nbt):
    src_r = edge_index[0].reshape(_NW, _KPW, _C)
    dst_r = edge_index[1].reshape(_NW, _KPW, _C)
    dst_r4 = edge_index[1].reshape(_NC, _NS, _KPT, _C)

    row = lambda v: v.reshape(1, _D)

    wcat = jnp.concatenate(
        [eW1[_D : 2 * _D], eW1[2 * _D :], nW1[:_D]], axis=1
    )
    pre = _precompute(x, wcat)
    xs1 = pre[:, :_D]
    xs2 = pre[:, _D : 2 * _D]
    xn = pre[:, 2 * _D :]

    pa = _gather_sum(xs1, xs2, src_r, dst_r)

    new_edge = _edge_mlp(
        edge_attr, pa, eW1[:_D], eW2, eW3, row(eb1), row(eb2), row(eb3),
        row(eg), row(ebt)
    )

    zeros_nd = jnp.zeros((_NP, _D), jnp.float32)
    agg2 = _segment_sum(new_edge, dst_r4, zeros_nd)

    x_out = _node_mlp(
        x, xn, agg2[:_N], agg2[_NP : _NP + _N], nW1[_D:], nW2, nW3, row(nb1),
        row(nb2), row(nb3), row(ng), row(nbt)
    )
    return (x_out, new_edge)


# speedup vs baseline: 4.5523x; 1.1005x over previous
import functools

import jax
import jax.numpy as jnp
from jax import lax
from jax.experimental import pallas as pl
from jax.experimental.pallas import tpu as pltpu
from jax.experimental.pallas import tpu_sc as plsc

_N = 10000
_E = 320000
_D = 128

_NC = 2
_NS = 16
_NW = _NC * _NS
_C = 80

_S = 5
_ES = _E // _S
_KPW = _ES // _NW // _C
_EPT = _ES // _NW
_KPT = _EPT // _C
_NP = 10240
_RPT = _NP // _NS


def _sc_mesh():
    return plsc.VectorSubcoreMesh(core_axis_name="c", subcore_axis_name="s")


def _gather_sum(xs1, xs2, src_r, dst_r):
    @functools.partial(
        pl.kernel,
        mesh=_sc_mesh(),
        out_type=jax.ShapeDtypeStruct((_ES, _D), jnp.float32),
        scratch_types=[
            pltpu.VMEM((_KPW, _C), jnp.int32),
            pltpu.VMEM((_KPW, _C), jnp.int32),
            pltpu.VMEM((_C, _D), jnp.float32),
            pltpu.VMEM((_C, _D), jnp.float32),
            pltpu.VMEM((_C, _D), jnp.float32),
            pltpu.VMEM((_C, _D), jnp.float32),
            pltpu.SemaphoreType.DMA,
            pltpu.SemaphoreType.DMA,
            pltpu.SemaphoreType.DMA,
            pltpu.SemaphoreType.DMA,
        ],
    )
    def k(xs1_h, xs2_h, src_h, dst_h, o_h, sidx, didx, a0, b0, a1, b1,
          g0, g1, o0, o1):
        wid = lax.axis_index("s") * _NC + lax.axis_index("c")
        kbase = wid * _KPW
        pltpu.sync_copy(src_h.at[wid], sidx)
        pltpu.sync_copy(dst_h.at[wid], didx)

        def start(i, a, b, g):
            pltpu.async_copy(xs1_h.at[sidx.at[i]], a, g)
            pltpu.async_copy(xs2_h.at[didx.at[i]], b, g)

        def fin(i, a, b, g, o):
            pltpu.make_async_copy(xs1_h.at[sidx.at[i]], a, g).wait()
            pltpu.make_async_copy(xs2_h.at[didx.at[i]], b, g).wait()

            def add_r(r, _):
                for j in range(8):
                    sl = pl.ds(j * 16, 16)
                    a[r, sl] = a[r, sl] + b[r, sl]
                return 0

            lax.fori_loop(0, _C, add_r, 0)
            pltpu.async_copy(a, o_h.at[pl.ds((kbase + i) * _C, _C)], o)

        def drain(i, a, o):
            pltpu.make_async_copy(a, o_h.at[pl.ds((kbase + i) * _C, _C)], o).wait()

        start(0, a0, b0, g0)

        def body(ii, _):
            i = 2 * ii

            @pl.when(i + 1 < _KPW)
            def _():
                @pl.when(i >= 1)
                def _():
                    drain(i - 1, a1, o1)

                start(i + 1, a1, b1, g1)

            fin(i, a0, b0, g0, o0)

            @pl.when(i + 2 < _KPW)
            def _():
                drain(i, a0, o0)
                start(i + 2, a0, b0, g0)

            @pl.when(i + 1 < _KPW)
            def _():
                fin(i + 1, a1, b1, g1, o1)

            return 0

        lax.fori_loop(0, (_KPW + 1) // 2, body, 0)
        drain(_KPW - 2, a1, o1)
        drain(_KPW - 1, a0, o0)

    return k(xs1, xs2, src_r, dst_r)


def _segment_sum(ne_slices, dst_slices, zer):
    r = len(ne_slices)

    @functools.partial(
        pl.kernel,
        mesh=_sc_mesh(),
        out_type=jax.ShapeDtypeStruct((_NC * _NP, _D), jnp.float32),
        scratch_types=(
            [pltpu.VMEM((_KPT, _C), jnp.int32) for _ in range(r)]
            + [
                pltpu.VMEM((_C, _D), jnp.float32),
                pltpu.VMEM((_C, _D), jnp.float32),
                pltpu.VMEM_SHARED((_NP, _D), jnp.float32),
                pltpu.SemaphoreType.DMA,
                pltpu.SemaphoreType.DMA,
            ]
        ),
    )
    def k(*refs):
        ne_h = refs[:r]
        dst_h = refs[r : 2 * r]
        zer_h = refs[2 * r]
        out_h = refs[2 * r + 1]
        didx = refs[2 * r + 2 : 3 * r + 2]
        rows0, rows1, agg_sh, g0, g1 = refs[3 * r + 2 :]

        c = lax.axis_index("c")
        s = lax.axis_index("s")
        rbase = s * _RPT
        pltpu.sync_copy(zer_h, agg_sh.at[pl.ds(rbase, _RPT)])
        for ss in range(r):
            pltpu.sync_copy(dst_h[ss].at[c, s], didx[ss])
        plsc.subcore_barrier()

        ebase = (c * _NS + s) * _EPT
        for ss in range(r):

            def start(i, rows, g, ss=ss):
                pltpu.async_copy(
                    ne_h[ss].at[pl.ds(ebase + i * _C, _C)], rows, g
                )

            def fin(i, rows, g, ss=ss):
                pltpu.make_async_copy(
                    ne_h[ss].at[pl.ds(ebase + i * _C, _C)], rows, g
                ).wait()
                pltpu.sync_copy(rows, agg_sh.at[didx[ss].at[i]], add=True)

            start(0, rows0, g0)

            def body(ii, _, start=start, fin=fin):
                i = 2 * ii

                @pl.when(i + 1 < _KPT)
                def _():
                    start(i + 1, rows1, g1)

                fin(i, rows0, g0)

                @pl.when(i + 2 < _KPT)
                def _():
                    start(i + 2, rows0, g0)

                @pl.when(i + 1 < _KPT)
                def _():
                    fin(i + 1, rows1, g1)

                return 0

            lax.fori_loop(0, (_KPT + 1) // 2, body, 0)

        plsc.subcore_barrier()
        pltpu.sync_copy(
            agg_sh.at[pl.ds(rbase, _RPT)], out_h.at[pl.ds(c * _NP + rbase, _RPT)]
        )

    return k(*ne_slices, *dst_slices, zer)


def _precompute(x, w1b, w1c, wn1a):
    def body(x_ref, wa_ref, wb_ref, wc_ref, oa_ref, ob_ref, oc_ref):
        xv = x_ref[...]
        oa_ref[...] = jnp.dot(xv, wa_ref[...], preferred_element_type=jnp.float32)
        ob_ref[...] = jnp.dot(xv, wb_ref[...], preferred_element_type=jnp.float32)
        oc_ref[...] = jnp.dot(xv, wc_ref[...], preferred_element_type=jnp.float32)

    out = jax.ShapeDtypeStruct((_N, _D), jnp.float32)
    return pl.pallas_call(body, out_shape=(out, out, out))(x, w1b, w1c, wn1a)


def _mlp_ln(h, w2_ref, w3_ref, b2_ref, b3_ref, g_ref, bt_ref):
    h = jnp.maximum(
        jnp.dot(h, w2_ref[...], preferred_element_type=jnp.float32) + b2_ref[...],
        0.0,
    )
    h = jnp.dot(h, w3_ref[...], preferred_element_type=jnp.float32) + b3_ref[...]
    m = jnp.mean(h, axis=1, keepdims=True)
    d = h - m
    v = jnp.mean(d * d, axis=1, keepdims=True)
    return d * lax.rsqrt(v + 1e-5) * g_ref[...] + bt_ref[...]


def _edge_mlp(ea_full, pa, sl, w1a, w2, w3, b1, b2, b3, g, bt):
    be = 4000
    nb = _ES // be

    def body(ea_ref, pa_ref, w1_ref, w2_ref, w3_ref, b1_ref, b2_ref,
             b3_ref, g_ref, bt_ref, o_ref):
        h = (
            jnp.dot(ea_ref[...], w1_ref[...], preferred_element_type=jnp.float32)
            + pa_ref[...]
            + b1_ref[...]
        )
        h = jnp.maximum(h, 0.0)
        o_ref[...] = _mlp_ln(h, w2_ref, w3_ref, b2_ref, b3_ref, g_ref, bt_ref)

    full = pl.BlockSpec((_D, _D), lambda i: (0, 0))
    vec = pl.BlockSpec((1, _D), lambda i: (0, 0))
    return pl.pallas_call(
        body,
        grid=(nb,),
        in_specs=[
            pl.BlockSpec((be, _D), lambda i, s0=sl: (s0 * nb + i, 0)),
            pl.BlockSpec((be, _D), lambda i: (i, 0)),
            full, full, full, vec, vec, vec, vec, vec,
        ],
        out_specs=pl.BlockSpec((be, _D), lambda i: (i, 0)),
        out_shape=jax.ShapeDtypeStruct((_ES, _D), jnp.float32),
    )(ea_full, pa, w1a, w2, w3, b1, b2, b3, g, bt)


def _node_mlp(x, xn, aggs, w1b, w2, w3, b1, b2, b3, g, bt):
    bn = 1000
    na = len(aggs)

    def body(*refs):
        x_ref, xn_ref = refs[0], refs[1]
        a_refs = refs[2 : 2 + na]
        w1_ref, w2_ref, w3_ref, b1_ref, b2_ref, b3_ref, g_ref, bt_ref, o_ref = (
            refs[2 + na :]
        )
        agg = a_refs[0][...]
        for a in a_refs[1:]:
            agg = agg + a[...]
        h = (
            jnp.dot(agg, w1_ref[...], preferred_element_type=jnp.float32)
            + xn_ref[...]
            + b1_ref[...]
        )
        h = jnp.maximum(h, 0.0)
        ln = _mlp_ln(h, w2_ref, w3_ref, b2_ref, b3_ref, g_ref, bt_ref)
        o_ref[...] = x_ref[...] + ln

    blk = pl.BlockSpec((bn, _D), lambda i: (i, 0))
    full = pl.BlockSpec((_D, _D), lambda i: (0, 0))
    vec = pl.BlockSpec((1, _D), lambda i: (0, 0))
    return pl.pallas_call(
        body,
        grid=(_N // bn,),
        in_specs=[blk, blk] + [blk] * na + [full, full, full, vec, vec, vec,
                                            vec, vec],
        out_specs=blk,
        out_shape=jax.ShapeDtypeStruct((_N, _D), jnp.float32),
    )(x, xn, *aggs, w1b, w2, w3, b1, b2, b3, g, bt)


def kernel(x, edge_attr, edge_index, eW1, eb1, eW2, eb2, eW3, eb3, eg, ebt,
           nW1, nb1, nW2, nb2, nW3, nb3, ng, nbt):
    src = edge_index[0]
    dst = edge_index[1]
    row = lambda v: v.reshape(1, _D)

    xs1, xs2, xn = _precompute(x, eW1[_D : 2 * _D], eW1[2 * _D :], nW1[:_D])

    ne = []
    for s in range(_S):
        lo = s * _ES
        src_r = lax.dynamic_slice_in_dim(src, lo, _ES).reshape(_NW, _KPW, _C)
        dst_r = lax.dynamic_slice_in_dim(dst, lo, _ES).reshape(_NW, _KPW, _C)
        pa = _gather_sum(xs1, xs2, src_r, dst_r)
        ne.append(
            _edge_mlp(edge_attr, pa, s, eW1[:_D], eW2, eW3, row(eb1),
                      row(eb2), row(eb3), row(eg), row(ebt))
        )

    zer = jnp.zeros((_RPT, _D), jnp.float32)
    dst4 = [
        lax.dynamic_slice_in_dim(dst, s * _ES, _ES).reshape(_NC, _NS, _KPT, _C)
        for s in range(_S)
    ]
    agg_a = _segment_sum(ne[:3], dst4[:3], zer)
    agg_b = _segment_sum(ne[3:], dst4[3:], zer)

    new_edge = jnp.concatenate(ne, axis=0)

    aggs = [agg_a[:_N], agg_a[_NP : _NP + _N], agg_b[:_N], agg_b[_NP : _NP + _N]]
    x_out = _node_mlp(x, xn, aggs, nW1[_D:], nW2, nW3, row(nb1), row(nb2),
                      row(nb3), row(ng), row(nbt))
    return (x_out, new_edge)


# speedup vs baseline: 4.7197x; 1.0368x over previous
import functools

import jax
import jax.numpy as jnp
from jax import lax
from jax.experimental import pallas as pl
from jax.experimental.pallas import tpu as pltpu
from jax.experimental.pallas import tpu_sc as plsc

_N = 10000
_E = 320000
_D = 128

_NC = 2
_NS = 16
_NW = _NC * _NS
_C = 80

_S = 5
_ES = _E // _S
_KPW = _ES // _NW // _C
_EPT = _ES // _NW
_KPT = _EPT // _C
_NP = 10240
_RPT = _NP // _NS


def _sc_mesh():
    return plsc.VectorSubcoreMesh(core_axis_name="c", subcore_axis_name="s")


def _gather_sum(xs1, xs2, src_r, dst_r):
    @functools.partial(
        pl.kernel,
        mesh=_sc_mesh(),
        out_type=jax.ShapeDtypeStruct((_ES, _D), jnp.float32),
        scratch_types=[
            pltpu.VMEM((_KPW, _C), jnp.int32),
            pltpu.VMEM((_KPW, _C), jnp.int32),
            pltpu.VMEM((_C, _D), jnp.float32),
            pltpu.VMEM((_C, _D), jnp.float32),
            pltpu.VMEM((_C, _D), jnp.float32),
            pltpu.VMEM((_C, _D), jnp.float32),
            pltpu.SemaphoreType.DMA,
            pltpu.SemaphoreType.DMA,
            pltpu.SemaphoreType.DMA,
            pltpu.SemaphoreType.DMA,
        ],
    )
    def k(xs1_h, xs2_h, src_h, dst_h, o_h, sidx, didx, a0, b0, a1, b1,
          g0, g1, o0, o1):
        wid = lax.axis_index("s") * _NC + lax.axis_index("c")
        kbase = wid * _KPW
        pltpu.sync_copy(src_h.at[wid], sidx)
        pltpu.sync_copy(dst_h.at[wid], didx)

        def start(i, a, b, g):
            pltpu.async_copy(xs1_h.at[sidx.at[i]], a, g)
            pltpu.async_copy(xs2_h.at[didx.at[i]], b, g)

        def fin(i, a, b, g, o):
            pltpu.make_async_copy(xs1_h.at[sidx.at[i]], a, g).wait()
            pltpu.make_async_copy(xs2_h.at[didx.at[i]], b, g).wait()

            def add_r(r, _):
                for j in range(8):
                    sl = pl.ds(j * 16, 16)
                    a[r, sl] = a[r, sl] + b[r, sl]
                return 0

            lax.fori_loop(0, _C, add_r, 0)
            pltpu.async_copy(a, o_h.at[pl.ds((kbase + i) * _C, _C)], o)

        def drain(i, a, b, o):
            del b
            pltpu.make_async_copy(a, o_h.at[pl.ds((kbase + i) * _C, _C)], o).wait()

        start(0, a0, b0, g0)

        def body(ii, _):
            i = 2 * ii

            @pl.when(i + 1 < _KPW)
            def _():
                @pl.when(i >= 1)
                def _():
                    drain(i - 1, a1, b1, o1)

                start(i + 1, a1, b1, g1)

            fin(i, a0, b0, g0, o0)

            @pl.when(i + 2 < _KPW)
            def _():
                drain(i, a0, b0, o0)
                start(i + 2, a0, b0, g0)

            @pl.when(i + 1 < _KPW)
            def _():
                fin(i + 1, a1, b1, g1, o1)

            return 0

        lax.fori_loop(0, (_KPW + 1) // 2, body, 0)
        drain(_KPW - 2, a1, b1, o1)
        drain(_KPW - 1, a0, b0, o0)

    return k(xs1, xs2, src_r, dst_r)


def _segment_sum(ne_slices, dst_slices, zer):
    r = len(ne_slices)

    @functools.partial(
        pl.kernel,
        mesh=_sc_mesh(),
        out_type=jax.ShapeDtypeStruct((_NC * _NP, _D), jnp.float32),
        scratch_types=(
            [pltpu.VMEM((_KPT, _C), jnp.int32) for _ in range(r)]
            + [
                pltpu.VMEM((_C, _D), jnp.float32),
                pltpu.VMEM((_C, _D), jnp.float32),
                pltpu.VMEM_SHARED((_NP, _D), jnp.float32),
                pltpu.SemaphoreType.DMA,
                pltpu.SemaphoreType.DMA,
            ]
        ),
    )
    def k(*refs):
        ne_h = refs[:r]
        dst_h = refs[r : 2 * r]
        zer_h = refs[2 * r]
        out_h = refs[2 * r + 1]
        didx = refs[2 * r + 2 : 3 * r + 2]
        rows0, rows1, agg_sh, g0, g1 = refs[3 * r + 2 :]

        c = lax.axis_index("c")
        s = lax.axis_index("s")
        rbase = s * _RPT
        pltpu.sync_copy(zer_h, agg_sh.at[pl.ds(rbase, _RPT)])
        for ss in range(r):
            pltpu.sync_copy(dst_h[ss].at[c, s], didx[ss])
        plsc.subcore_barrier()

        ebase = (c * _NS + s) * _EPT
        for ss in range(r):

            def start(i, rows, g, ss=ss):
                pltpu.async_copy(
                    ne_h[ss].at[pl.ds(ebase + i * _C, _C)], rows, g
                )

            def fin(i, rows, g, ss=ss):
                pltpu.make_async_copy(
                    ne_h[ss].at[pl.ds(ebase + i * _C, _C)], rows, g
                ).wait()
                pltpu.sync_copy(rows, agg_sh.at[didx[ss].at[i]], add=True)

            start(0, rows0, g0)

            def body(ii, _, start=start, fin=fin):
                i = 2 * ii

                @pl.when(i + 1 < _KPT)
                def _():
                    start(i + 1, rows1, g1)

                fin(i, rows0, g0)

                @pl.when(i + 2 < _KPT)
                def _():
                    start(i + 2, rows0, g0)

                @pl.when(i + 1 < _KPT)
                def _():
                    fin(i + 1, rows1, g1)

                return 0

            lax.fori_loop(0, (_KPT + 1) // 2, body, 0)

        plsc.subcore_barrier()
        pltpu.sync_copy(
            agg_sh.at[pl.ds(rbase, _RPT)], out_h.at[pl.ds(c * _NP + rbase, _RPT)]
        )

    return k(*ne_slices, *dst_slices, zer)


def _precompute(x, w1b, w1c, wn1a):
    def body(x_ref, wa_ref, wb_ref, wc_ref, oa_ref, ob_ref, oc_ref):
        xv = x_ref[...]
        oa_ref[...] = jnp.dot(xv, wa_ref[...], preferred_element_type=jnp.float32)
        ob_ref[...] = jnp.dot(xv, wb_ref[...], preferred_element_type=jnp.float32)
        oc_ref[...] = jnp.dot(xv, wc_ref[...], preferred_element_type=jnp.float32)

    out = jax.ShapeDtypeStruct((_N, _D), jnp.float32)
    return pl.pallas_call(body, out_shape=(out, out, out))(x, w1b, w1c, wn1a)


def _mlp_ln(h, w2_ref, w3_ref, b2_ref, b3_ref, g_ref, bt_ref):
    h = jnp.maximum(
        jnp.dot(h, w2_ref[...], preferred_element_type=jnp.float32) + b2_ref[...],
        0.0,
    )
    h = jnp.dot(h, w3_ref[...], preferred_element_type=jnp.float32) + b3_ref[...]
    m = jnp.mean(h, axis=1, keepdims=True)
    d = h - m
    v = jnp.mean(d * d, axis=1, keepdims=True)
    return d * lax.rsqrt(v + 1e-5) * g_ref[...] + bt_ref[...]


def _edge_mlp(ea_full, pa, sl, w1a, w2, w3, b1, b2, b3, g, bt, ne_full):
    be = 4000
    nb = _ES // be

    def body(*refs):
        (ea_ref, pa_ref, w1_ref, w2_ref, w3_ref, b1_ref, b2_ref, b3_ref,
         g_ref, bt_ref) = refs[:10]
        o_ref, of_ref = refs[-2:]
        h = (
            jnp.dot(ea_ref[...], w1_ref[...], preferred_element_type=jnp.float32)
            + pa_ref[...]
            + b1_ref[...]
        )
        h = jnp.maximum(h, 0.0)
        r = _mlp_ln(h, w2_ref, w3_ref, b2_ref, b3_ref, g_ref, bt_ref)
        o_ref[...] = r
        of_ref[...] = r

    full = pl.BlockSpec((_D, _D), lambda i: (0, 0))
    vec = pl.BlockSpec((1, _D), lambda i: (0, 0))
    in_specs = [
        pl.BlockSpec((be, _D), lambda i, s0=sl: (s0 * nb + i, 0)),
        pl.BlockSpec((be, _D), lambda i: (i, 0)),
        full, full, full, vec, vec, vec, vec, vec,
    ]
    args = [ea_full, pa, w1a, w2, w3, b1, b2, b3, g, bt]
    aliases = {}
    if ne_full is not None:
        in_specs.append(pl.BlockSpec(memory_space=pl.ANY))
        args.append(ne_full)
        aliases = {10: 1}
    return pl.pallas_call(
        body,
        grid=(nb,),
        in_specs=in_specs,
        out_specs=(
            pl.BlockSpec((be, _D), lambda i: (i, 0)),
            pl.BlockSpec((be, _D), lambda i, s0=sl: (s0 * nb + i, 0)),
        ),
        out_shape=(
            jax.ShapeDtypeStruct((_ES, _D), jnp.float32),
            jax.ShapeDtypeStruct((_E, _D), jnp.float32),
        ),
        input_output_aliases=aliases,
    )(*args)


def _node_mlp(x, xn, aggs, w1b, w2, w3, b1, b2, b3, g, bt):
    bn = 1000
    na = len(aggs)

    def body(*refs):
        x_ref, xn_ref = refs[0], refs[1]
        a_refs = refs[2 : 2 + na]
        w1_ref, w2_ref, w3_ref, b1_ref, b2_ref, b3_ref, g_ref, bt_ref, o_ref = (
            refs[2 + na :]
        )
        agg = a_refs[0][...]
        for a in a_refs[1:]:
            agg = agg + a[...]
        h = (
            jnp.dot(agg, w1_ref[...], preferred_element_type=jnp.float32)
            + xn_ref[...]
            + b1_ref[...]
        )
        h = jnp.maximum(h, 0.0)
        ln = _mlp_ln(h, w2_ref, w3_ref, b2_ref, b3_ref, g_ref, bt_ref)
        o_ref[...] = x_ref[...] + ln

    blk = pl.BlockSpec((bn, _D), lambda i: (i, 0))
    full = pl.BlockSpec((_D, _D), lambda i: (0, 0))
    vec = pl.BlockSpec((1, _D), lambda i: (0, 0))
    return pl.pallas_call(
        body,
        grid=(_N // bn,),
        in_specs=[blk, blk] + [blk] * na + [full, full, full, vec, vec, vec,
                                            vec, vec],
        out_specs=blk,
        out_shape=jax.ShapeDtypeStruct((_N, _D), jnp.float32),
    )(x, xn, *aggs, w1b, w2, w3, b1, b2, b3, g, bt)


def kernel(x, edge_attr, edge_index, eW1, eb1, eW2, eb2, eW3, eb3, eg, ebt,
           nW1, nb1, nW2, nb2, nW3, nb3, ng, nbt):
    src = edge_index[0]
    dst = edge_index[1]
    row = lambda v: v.reshape(1, _D)

    xs1, xs2, xn = _precompute(x, eW1[_D : 2 * _D], eW1[2 * _D :], nW1[:_D])

    ne = []
    ne_full = None
    for s in range(_S):
        lo = s * _ES
        src_r = lax.dynamic_slice_in_dim(src, lo, _ES).reshape(_NW, _KPW, _C)
        dst_r = lax.dynamic_slice_in_dim(dst, lo, _ES).reshape(_NW, _KPW, _C)
        pa = _gather_sum(xs1, xs2, src_r, dst_r)
        ne_s, ne_full = _edge_mlp(
            edge_attr, pa, s, eW1[:_D], eW2, eW3, row(eb1), row(eb2),
            row(eb3), row(eg), row(ebt), ne_full
        )
        ne.append(ne_s)

    zer = jnp.zeros((_RPT, _D), jnp.float32)
    dst4 = [
        lax.dynamic_slice_in_dim(dst, s * _ES, _ES).reshape(_NC, _NS, _KPT, _C)
        for s in range(_S)
    ]
    agg_a = _segment_sum(ne[:3], dst4[:3], zer)
    agg_b = _segment_sum(ne[3:], dst4[3:], zer)

    new_edge = ne_full

    aggs = [agg_a[:_N], agg_a[_NP : _NP + _N], agg_b[:_N], agg_b[_NP : _NP + _N]]
    x_out = _node_mlp(x, xn, aggs, nW1[_D:], nW2, nW3, row(nb1), row(nb2),
                      row(nb3), row(ng), row(nbt))
    return (x_out, new_edge)


# speedup vs baseline: 4.8226x; 1.0218x over previous
import functools

import jax
import jax.numpy as jnp
from jax import lax
from jax.experimental import pallas as pl
from jax.experimental.pallas import tpu as pltpu
from jax.experimental.pallas import tpu_sc as plsc

_N = 10000
_E = 320000
_D = 128

_NC = 2
_NS = 16
_NW = _NC * _NS
_C = 80

_S = 5
_ES = _E // _S
_KPW = _ES // _NW // _C
_EPT = _ES // _NW
_KPT = _EPT // _C
_NP = 10240
_RPT = _NP // _NS


def _sc_mesh():
    return plsc.VectorSubcoreMesh(core_axis_name="c", subcore_axis_name="s")


def _gather_sum(xs1, xs2, src_r, dst_r):
    @functools.partial(
        pl.kernel,
        mesh=_sc_mesh(),
        out_type=jax.ShapeDtypeStruct((_ES, _D), jnp.float32),
        scratch_types=[
            pltpu.VMEM((_KPW, _C), jnp.int32),
            pltpu.VMEM((_KPW, _C), jnp.int32),
            pltpu.VMEM((_C, _D), jnp.float32),
            pltpu.VMEM((_C, _D), jnp.float32),
            pltpu.VMEM((_C, _D), jnp.float32),
            pltpu.VMEM((_C, _D), jnp.float32),
            pltpu.SemaphoreType.DMA,
            pltpu.SemaphoreType.DMA,
            pltpu.SemaphoreType.DMA,
            pltpu.SemaphoreType.DMA,
        ],
    )
    def k(xs1_h, xs2_h, src_h, dst_h, o_h, sidx, didx, a0, b0, a1, b1,
          g0, g1, o0, o1):
        wid = lax.axis_index("s") * _NC + lax.axis_index("c")
        kbase = wid * _KPW
        pltpu.sync_copy(src_h.at[wid], sidx)
        pltpu.sync_copy(dst_h.at[wid], didx)

        def start(i, a, b, g):
            pltpu.async_copy(xs1_h.at[sidx.at[i]], a, g)
            pltpu.async_copy(xs2_h.at[didx.at[i]], b, g)

        def fin(i, a, b, g, o):
            pltpu.make_async_copy(xs1_h.at[sidx.at[i]], a, g).wait()
            pltpu.make_async_copy(xs2_h.at[didx.at[i]], b, g).wait()

            def add_r(r, _):
                for j in range(8):
                    sl = pl.ds(j * 16, 16)
                    a[r, sl] = a[r, sl] + b[r, sl]
                return 0

            lax.fori_loop(0, _C, add_r, 0)
            pltpu.async_copy(a, o_h.at[pl.ds((kbase + i) * _C, _C)], o)

        def drain(i, a, b, o):
            del b
            pltpu.make_async_copy(a, o_h.at[pl.ds((kbase + i) * _C, _C)], o).wait()

        start(0, a0, b0, g0)

        def body(ii, _):
            i = 2 * ii

            @pl.when(i + 1 < _KPW)
            def _():
                @pl.when(i >= 1)
                def _():
                    drain(i - 1, a1, b1, o1)

                start(i + 1, a1, b1, g1)

            fin(i, a0, b0, g0, o0)

            @pl.when(i + 2 < _KPW)
            def _():
                drain(i, a0, b0, o0)
                start(i + 2, a0, b0, g0)

            @pl.when(i + 1 < _KPW)
            def _():
                fin(i + 1, a1, b1, g1, o1)

            return 0

        lax.fori_loop(0, (_KPW + 1) // 2, body, 0)
        drain(_KPW - 2, a1, b1, o1)
        drain(_KPW - 1, a0, b0, o0)

    return k(xs1, xs2, src_r, dst_r)


def _segment_sum(ne_slices, dst_slices):
    r = len(ne_slices)

    @functools.partial(
        pl.kernel,
        mesh=_sc_mesh(),
        out_type=jax.ShapeDtypeStruct((_NC * _NP, _D), jnp.float32),
        scratch_types=(
            [pltpu.VMEM((_KPT, _C), jnp.int32) for _ in range(r)]
            + [
                pltpu.VMEM((_C, _D), jnp.float32),
                pltpu.VMEM((_C, _D), jnp.float32),
                pltpu.VMEM_SHARED((_NP, _D), jnp.float32),
                pltpu.SemaphoreType.DMA,
                pltpu.SemaphoreType.DMA,
                pltpu.SemaphoreType.DMA,
                pltpu.SemaphoreType.DMA,
            ]
        ),
    )
    def k(*refs):
        ne_h = refs[:r]
        dst_h = refs[r : 2 * r]
        out_h = refs[2 * r]
        didx = refs[2 * r + 1 : 3 * r + 1]
        rows0, rows1, agg_sh, g0, g1, s0, s1 = refs[3 * r + 1 :]

        c = lax.axis_index("c")
        s = lax.axis_index("s")
        rbase = s * _RPT

        def zrow(rr, _):
            for j in range(8):
                rows0[rr, pl.ds(j * 16, 16)] = jnp.zeros((16,), jnp.float32)
            return 0

        lax.fori_loop(0, _C, zrow, 0)
        for z in range(_RPT // _C):
            pltpu.sync_copy(rows0, agg_sh.at[pl.ds(rbase + z * _C, _C)])
        for ss in range(r):
            pltpu.sync_copy(dst_h[ss].at[c, s], didx[ss])
        plsc.subcore_barrier()

        ebase = (c * _NS + s) * _EPT
        for ss in range(r):

            def load(i, rows, g, ss=ss):
                pltpu.async_copy(
                    ne_h[ss].at[pl.ds(ebase + i * _C, _C)], rows, g
                )

            def loadwait(i, rows, g, ss=ss):
                pltpu.make_async_copy(
                    ne_h[ss].at[pl.ds(ebase + i * _C, _C)], rows, g
                ).wait()

            def scat(i, rows, sm, ss=ss):
                pltpu.async_copy(rows, agg_sh.at[didx[ss].at[i]], sm)

            def scatwait(i, rows, sm, ss=ss):
                pltpu.make_async_copy(
                    rows, agg_sh.at[didx[ss].at[i]], sm
                ).wait()

            load(0, rows0, g0)

            def body(ii, _, load=load, loadwait=loadwait, scat=scat,
                     scatwait=scatwait):
                i = 2 * ii

                @pl.when(i + 1 < _KPT)
                def _():
                    @pl.when(i >= 1)
                    def _():
                        scatwait(i - 1, rows1, s1)

                    load(i + 1, rows1, g1)

                loadwait(i, rows0, g0)
                scat(i, rows0, s0)

                @pl.when(i + 2 < _KPT)
                def _():
                    scatwait(i, rows0, s0)
                    load(i + 2, rows0, g0)

                @pl.when(i + 1 < _KPT)
                def _():
                    loadwait(i + 1, rows1, g1)
                    scat(i + 1, rows1, s1)

                return 0

            lax.fori_loop(0, (_KPT + 1) // 2, body, 0)
            scatwait(_KPT - 2, rows1, s1)
            scatwait(_KPT - 1, rows0, s0)

        plsc.subcore_barrier()
        pltpu.sync_copy(
            agg_sh.at[pl.ds(rbase, _RPT)], out_h.at[pl.ds(c * _NP + rbase, _RPT)]
        )

    return k(*ne_slices, *dst_slices)


def _precompute(x, w1b, w1c, wn1a):
    def body(x_ref, wa_ref, wb_ref, wc_ref, oa_ref, ob_ref, oc_ref):
        xv = x_ref[...]
        oa_ref[...] = jnp.dot(xv, wa_ref[...], preferred_element_type=jnp.float32)
        ob_ref[...] = jnp.dot(xv, wb_ref[...], preferred_element_type=jnp.float32)
        oc_ref[...] = jnp.dot(xv, wc_ref[...], preferred_element_type=jnp.float32)

    out = jax.ShapeDtypeStruct((_N, _D), jnp.float32)
    return pl.pallas_call(body, out_shape=(out, out, out))(x, w1b, w1c, wn1a)


def _mlp_ln(h, w2_ref, w3_ref, b2_ref, b3_ref, g_ref, bt_ref):
    h = jnp.maximum(
        jnp.dot(h, w2_ref[...], preferred_element_type=jnp.float32) + b2_ref[...],
        0.0,
    )
    h = jnp.dot(h, w3_ref[...], preferred_element_type=jnp.float32) + b3_ref[...]
    m = jnp.mean(h, axis=1, keepdims=True)
    d = h - m
    v = jnp.mean(d * d, axis=1, keepdims=True)
    return d * lax.rsqrt(v + 1e-5) * g_ref[...] + bt_ref[...]


def _edge_mlp(ea_full, pa, sl, w1a, w2, w3, b1, b2, b3, g, bt, ne_full):
    be = 4000
    nb = _ES // be

    def body(*refs):
        (ea_ref, pa_ref, w1_ref, w2_ref, w3_ref, b1_ref, b2_ref, b3_ref,
         g_ref, bt_ref) = refs[:10]
        o_ref, of_ref = refs[-2:]
        h = (
            jnp.dot(ea_ref[...], w1_ref[...], preferred_element_type=jnp.float32)
            + pa_ref[...]
            + b1_ref[...]
        )
        h = jnp.maximum(h, 0.0)
        r = _mlp_ln(h, w2_ref, w3_ref, b2_ref, b3_ref, g_ref, bt_ref)
        o_ref[...] = r
        of_ref[...] = r

    full = pl.BlockSpec((_D, _D), lambda i: (0, 0))
    vec = pl.BlockSpec((1, _D), lambda i: (0, 0))
    in_specs = [
        pl.BlockSpec((be, _D), lambda i, s0=sl: (s0 * nb + i, 0)),
        pl.BlockSpec((be, _D), lambda i: (i, 0)),
        full, full, full, vec, vec, vec, vec, vec,
    ]
    args = [ea_full, pa, w1a, w2, w3, b1, b2, b3, g, bt]
    aliases = {}
    if ne_full is not None:
        in_specs.append(pl.BlockSpec(memory_space=pl.ANY))
        args.append(ne_full)
        aliases = {10: 1}
    return pl.pallas_call(
        body,
        grid=(nb,),
        in_specs=in_specs,
        out_specs=(
            pl.BlockSpec((be, _D), lambda i: (i, 0)),
            pl.BlockSpec((be, _D), lambda i, s0=sl: (s0 * nb + i, 0)),
        ),
        out_shape=(
            jax.ShapeDtypeStruct((_ES, _D), jnp.float32),
            jax.ShapeDtypeStruct((_E, _D), jnp.float32),
        ),
        input_output_aliases=aliases,
    )(*args)


def _node_mlp(x, xn, aggs, w1b, w2, w3, b1, b2, b3, g, bt):
    bn = 1000
    na = len(aggs)

    def body(*refs):
        x_ref, xn_ref = refs[0], refs[1]
        a_refs = refs[2 : 2 + na]
        w1_ref, w2_ref, w3_ref, b1_ref, b2_ref, b3_ref, g_ref, bt_ref, o_ref = (
            refs[2 + na :]
        )
        agg = a_refs[0][...]
        for a in a_refs[1:]:
            agg = agg + a[...]
        h = (
            jnp.dot(agg, w1_ref[...], preferred_element_type=jnp.float32)
            + xn_ref[...]
            + b1_ref[...]
        )
        h = jnp.maximum(h, 0.0)
        ln = _mlp_ln(h, w2_ref, w3_ref, b2_ref, b3_ref, g_ref, bt_ref)
        o_ref[...] = x_ref[...] + ln

    blk = pl.BlockSpec((bn, _D), lambda i: (i, 0))
    full = pl.BlockSpec((_D, _D), lambda i: (0, 0))
    vec = pl.BlockSpec((1, _D), lambda i: (0, 0))
    return pl.pallas_call(
        body,
        grid=(_N // bn,),
        in_specs=[blk, blk] + [blk] * na + [full, full, full, vec, vec, vec,
                                            vec, vec],
        out_specs=blk,
        out_shape=jax.ShapeDtypeStruct((_N, _D), jnp.float32),
    )(x, xn, *aggs, w1b, w2, w3, b1, b2, b3, g, bt)


def kernel(x, edge_attr, edge_index, eW1, eb1, eW2, eb2, eW3, eb3, eg, ebt,
           nW1, nb1, nW2, nb2, nW3, nb3, ng, nbt):
    src = edge_index[0]
    dst = edge_index[1]
    row = lambda v: v.reshape(1, _D)

    xs1, xs2, xn = _precompute(x, eW1[_D : 2 * _D], eW1[2 * _D :], nW1[:_D])

    ne = []
    ne_full = None
    for s in range(_S):
        lo = s * _ES
        src_r = lax.dynamic_slice_in_dim(src, lo, _ES).reshape(_NW, _KPW, _C)
        dst_r = lax.dynamic_slice_in_dim(dst, lo, _ES).reshape(_NW, _KPW, _C)
        pa = _gather_sum(xs1, xs2, src_r, dst_r)
        ne_s, ne_full = _edge_mlp(
            edge_attr, pa, s, eW1[:_D], eW2, eW3, row(eb1), row(eb2),
            row(eb3), row(eg), row(ebt), ne_full
        )
        ne.append(ne_s)

    dst4 = [
        lax.dynamic_slice_in_dim(dst, s * _ES, _ES).reshape(_NC, _NS, _KPT, _C)
        for s in range(_S)
    ]
    agg_a = _segment_sum(ne[:3], dst4[:3])
    agg_b = _segment_sum(ne[3:], dst4[3:])

    new_edge = ne_full

    aggs = [agg_a[:_N], agg_a[_NP : _NP + _N], agg_b[:_N], agg_b[_NP : _NP + _N]]
    x_out = _node_mlp(x, xn, aggs, nW1[_D:], nW2, nW3, row(nb1), row(nb2),
                      row(nb3), row(ng), row(nbt))
    return (x_out, new_edge)
